# baseline (speedup 1.0000x reference)
import functools

import jax
import jax.numpy as jnp
from jax import lax
from jax.experimental import pallas as pl
from jax.experimental.pallas import tpu as pltpu
from jax.experimental.pallas import tpu_sc as plsc

NN = 10000
EE = 320000
DD = 128
HH = 32
GG = 64
NP = 10240
CH = 128
NBLK = EE // CH
NWORK = 32
PERT = NP // 16

_mesh = plsc.VectorSubcoreMesh(core_axis_name="c", subcore_axis_name="s")


def _wid(c, s):
    return s * 2 + c


def _nblk(wid):
    base = NBLK // NWORK
    rem = NBLK % NWORK
    return base + (wid < rem).astype(jnp.int32)


@functools.partial(
    pl.kernel,
    out_type=jax.ShapeDtypeStruct((2, NP), jnp.float32),
    mesh=_mesh,
    scratch_types=[
        pltpu.VMEM((CH,), jnp.int32),
        pltpu.VMEM((CH,), jnp.float32),
        pltpu.VMEM((PERT,), jnp.float32),
        pltpu.VMEM_SHARED((NP,), jnp.float32),
    ],
)
def _k1_deg(dst_hbm, z640_hbm, deg_out, idx_d, ones_v, zb, deg_s):
    c = lax.axis_index("c")
    s = lax.axis_index("s")
    wid = _wid(c, s)
    for i in range(CH // 16):
        ones_v[pl.ds(16 * i, 16)] = jnp.full((16,), 1.0, jnp.float32)
    pltpu.sync_copy(z640_hbm, zb)
    pltpu.sync_copy(zb, deg_s.at[pl.ds(s * PERT, PERT)])
    plsc.subcore_barrier()

    def body(j, carry):
        off = (wid + NWORK * j) * CH
        pltpu.sync_copy(dst_hbm.at[pl.ds(off, CH)], idx_d)
        pltpu.sync_copy(ones_v, deg_s.at[idx_d], add=True)
        return carry

    lax.fori_loop(0, _nblk(wid), body, 0)
    plsc.subcore_barrier()
    pltpu.sync_copy(deg_s.at[pl.ds(s * PERT, PERT)],
                    deg_out.at[c].at[pl.ds(s * PERT, PERT)])


def _k2a_body(d0_ref, d1_ref, disrepT_ref):
    deg = d0_ref[0] + d1_ref[0] + 1.0
    dis = 1.0 / jnp.sqrt(deg)
    disrepT_ref[...] = jnp.broadcast_to(dis, (HH, dis.shape[1]))


def _k2a(deg_p):
    rb = 1024
    grid = NP // rb
    deg3 = deg_p.reshape(2 * NP // rb, 1, rb)
    return pl.pallas_call(
        _k2a_body,
        grid=(grid,),
        in_specs=[
            pl.BlockSpec((1, 1, rb), lambda i: (i, 0, 0)),
            pl.BlockSpec((1, 1, rb), lambda i: (i + NP // rb, 0, 0)),
        ],
        out_specs=pl.BlockSpec((HH, rb), lambda i: (0, i)),
        out_shape=jax.ShapeDtypeStruct((HH, NP), jnp.float32),
    )(deg3, deg3)


def _k2b_body(x_ref, w1_ref, disrep_ref, hd_ref):
    h = jnp.dot(x_ref[...], w1_ref[...], preferred_element_type=jnp.float32)
    hd_ref[...] = h * disrep_ref[...]


def _k2b(x_p, w1, disrep):
    rb = 1024
    grid = NP // rb
    return pl.pallas_call(
        _k2b_body,
        grid=(grid,),
        in_specs=[
            pl.BlockSpec((rb, DD), lambda i: (i, 0)),
            pl.BlockSpec((DD, HH), lambda i: (0, 0)),
            pl.BlockSpec((rb, HH), lambda i: (i, 0)),
        ],
        out_specs=pl.BlockSpec((rb, HH), lambda i: (i, 0)),
        out_shape=jax.ShapeDtypeStruct((NP, HH), jnp.float32),
    )(x_p, w1, disrep)


@functools.partial(
    pl.kernel,
    out_type=jax.ShapeDtypeStruct((2, NP, HH), jnp.float32),
    mesh=_mesh,
    scratch_types=[
        pltpu.VMEM((CH,), jnp.int32),
        pltpu.VMEM((CH,), jnp.int32),
        pltpu.VMEM((CH, HH), jnp.float32),
        pltpu.VMEM((CH, HH), jnp.float32),
        pltpu.SemaphoreType.DMA,
        pltpu.VMEM_SHARED((NP, HH), jnp.float32),
    ],
    compiler_params=pltpu.CompilerParams(use_tc_tiling_on_sc=False),
)
def _k3_msg(hd_hbm, src_hbm, dst_hbm, zrow_hbm, acc_out,
            idx_s, idx_d, rows, zb, sem, acc_s):
    c = lax.axis_index("c")
    s = lax.axis_index("s")
    wid = _wid(c, s)
    pltpu.sync_copy(zrow_hbm, zb)
    for j in range(PERT // CH):
        pltpu.sync_copy(zb, acc_s.at[pl.ds(s * PERT + j * CH, CH)])
    plsc.subcore_barrier()

    def body(j, carry):
        off = (wid + NWORK * j) * CH
        pltpu.sync_copy(src_hbm.at[pl.ds(off, CH)], idx_s)
        pltpu.sync_copy(dst_hbm.at[pl.ds(off, CH)], idx_d)
        pltpu.async_copy(hd_hbm.at[idx_s], rows, sem).wait()
        pltpu.sync_copy(rows, acc_s.at[idx_d], add=True)
        return carry

    lax.fori_loop(0, _nblk(wid), body, 0)
    plsc.subcore_barrier()
    pltpu.sync_copy(acc_s.at[pl.ds(s * PERT, PERT)],
                    acc_out.at[c].at[pl.ds(s * PERT, PERT)])


def _k4_body(acc_ref, hd_ref, disrep_ref, b1_ref, w2_ref, zd_ref):
    disrep = disrep_ref[...]
    pre = (acc_ref[0] + acc_ref[1] + hd_ref[...]) * disrep + b1_ref[...]
    a1 = jnp.maximum(pre, 0.0)
    z = jnp.dot(a1, w2_ref[...], preferred_element_type=jnp.float32)
    zd_ref[...] = z * disrep


def _k4(acc_p, hd, disrep, b1r, w2):
    rb = 1024
    grid = NP // rb
    return pl.pallas_call(
        _k4_body,
        grid=(grid,),
        in_specs=[
            pl.BlockSpec((2, rb, HH), lambda i: (0, i, 0)),
            pl.BlockSpec((rb, HH), lambda i: (i, 0)),
            pl.BlockSpec((rb, HH), lambda i: (i, 0)),
            pl.BlockSpec((1, HH), lambda i: (0, 0)),
            pl.BlockSpec((HH, HH), lambda i: (0, 0)),
        ],
        out_specs=pl.BlockSpec((rb, HH), lambda i: (i, 0)),
        out_shape=jax.ShapeDtypeStruct((NP, HH), jnp.float32),
    )(acc_p, hd, disrep, b1r, w2)


@functools.partial(
    pl.kernel,
    out_type=(jax.ShapeDtypeStruct((2, CH, HH), jnp.float32),
              jax.ShapeDtypeStruct((2, CH, HH), jnp.float32)),
    mesh=_mesh,
    scratch_types=[
        pltpu.VMEM((CH,), jnp.int32),
        pltpu.VMEM((CH,), jnp.int32),
        pltpu.VMEM((CH, HH), jnp.float32),
        pltpu.VMEM((CH, HH), jnp.float32),
        pltpu.VMEM((CH, HH), jnp.float32),
        pltpu.VMEM((CH, HH), jnp.float32),
        pltpu.VMEM((CH, HH), jnp.float32),
        pltpu.VMEM((CH,), jnp.float32),
        pltpu.VMEM((CH,), jnp.int32),
        pltpu.VMEM((CH, HH), jnp.float32),
        pltpu.SemaphoreType.DMA,
        pltpu.VMEM_SHARED((NP, HH), jnp.float32),
        pltpu.VMEM_SHARED((CH, HH), jnp.float32),
        pltpu.VMEM_SHARED((CH, HH), jnp.float32),
    ],
    compiler_params=pltpu.CompilerParams(
        use_tc_tiling_on_sc=False, needs_layout_passes=False),
)
def _k5_pool(zd_hbm, dis_hbm, batch_hbm, src_hbm, dst_hbm, zrow_hbm,
             bins_out, cnt_out,
             idx_s, idx_d, rows, rbuf2, zbuf2, qbuf2, cbuf2, dbuf, bbuf, zb,
             sem, racc_s, bins_s, cbin_s):
    c = lax.axis_index("c")
    s = lax.axis_index("s")
    wid = _wid(c, s)
    pltpu.sync_copy(zrow_hbm, zb)
    for j in range(PERT // CH):
        pltpu.sync_copy(zb, racc_s.at[pl.ds(s * PERT + j * CH, CH)])

    @pl.when(s == 0)
    def _():
        pltpu.sync_copy(zb, bins_s)

    @pl.when(s == 1)
    def _():
        pltpu.sync_copy(zb, cbin_s)

    plsc.subcore_barrier()

    def body(j, carry):
        off = (wid + NWORK * j) * CH
        pltpu.sync_copy(src_hbm.at[pl.ds(off, CH)], idx_s)
        pltpu.sync_copy(dst_hbm.at[pl.ds(off, CH)], idx_d)
        pltpu.async_copy(zd_hbm.at[idx_s], rows, sem).wait()
        pltpu.sync_copy(rows, racc_s.at[idx_d], add=True)
        return carry

    lax.fori_loop(0, _nblk(wid), body, 0)
    plsc.subcore_barrier()

    coef = jnp.where(c == 0, 1.0, 0.0).astype(jnp.float32)

    def cfill(v, carry):
        for half in range(HH // 16):
            cbuf2[v, pl.ds(16 * half, 16)] = jnp.zeros((16,), jnp.float32) + coef
        return carry

    lax.fori_loop(0, CH, cfill, 0)

    for k in range(PERT // CH):
        off = s * PERT + k * CH
        pltpu.sync_copy(racc_s.at[pl.ds(off, CH)], rbuf2)
        pltpu.sync_copy(zd_hbm.at[pl.ds(off, CH)], zbuf2)
        pltpu.sync_copy(dis_hbm.at[pl.ds(off, CH)], dbuf)
        pltpu.sync_copy(batch_hbm.at[pl.ds(off, CH)], bbuf)

        def nbody(v, carry):
            dv = plsc.load_gather(dbuf, [jnp.zeros((16,), jnp.int32) + v])
            for half in range(HH // 16):
                sl = pl.ds(16 * half, 16)
                qbuf2[v, sl] = dv * (rbuf2[v, sl] + coef * zbuf2[v, sl])
            return carry

        lax.fori_loop(0, CH, nbody, 0)
        pltpu.sync_copy(qbuf2, bins_s.at[bbuf], add=True)
        pltpu.sync_copy(cbuf2, cbin_s.at[bbuf], add=True)
    plsc.subcore_barrier()

    @pl.when(s == 0)
    def _():
        pltpu.sync_copy(bins_s, bins_out.at[c])

    @pl.when(s == 1)
    def _():
        pltpu.sync_copy(cbin_s, cnt_out.at[c])


def _k6_body(bins_ref, cnt_ref, b2_ref, wl_ref, bl_ref, out_ref):
    b = bins_ref[0, :GG] + bins_ref[1, :GG]
    cn = cnt_ref[0, :GG] + cnt_ref[1, :GG]
    pooled = b / jnp.maximum(cn, 1.0) + jnp.where(cn > 0.0, b2_ref[...], 0.0)
    out = jnp.dot(pooled, wl_ref[...], preferred_element_type=jnp.float32)
    out_ref[...] = out + bl_ref[...]


def _k6(bins_p, cnt_p, b2r, wl, blr):
    return pl.pallas_call(
        _k6_body,
        in_specs=[
            pl.BlockSpec((2, CH, HH), lambda: (0, 0, 0)),
            pl.BlockSpec((2, CH, HH), lambda: (0, 0, 0)),
            pl.BlockSpec((1, HH), lambda: (0, 0)),
            pl.BlockSpec((HH, 1), lambda: (0, 0)),
            pl.BlockSpec((1, 1), lambda: (0, 0)),
        ],
        out_specs=pl.BlockSpec((GG, 1), lambda: (0, 0)),
        out_shape=jax.ShapeDtypeStruct((GG, 1), jnp.float32),
    )(bins_p, cnt_p, b2r, wl, blr)


def kernel(x, edge_index, batch, W1, b1, W2, b2, Wl, bl):
    src = edge_index[0]
    dst = edge_index[1]
    x_p = jnp.pad(x, ((0, NP - NN), (0, 0)))
    batch_p = jnp.pad(batch, (0, NP - NN), constant_values=GG)
    z640 = jnp.zeros((PERT,), jnp.float32)
    zrow = jnp.zeros((CH, HH), jnp.float32)
    b1r = b1.reshape(1, HH)
    b2r = b2.reshape(1, HH)
    blr = bl.reshape(1, 1)

    deg_p = _k1_deg(dst, z640)
    disrep = _k2a(deg_p).T
    hd = _k2b(x_p, W1, disrep)
    acc_p = _k3_msg(hd, src, dst, zrow)
    zdrep = _k4(acc_p, hd, disrep, b1r, W2)
    bins_p, cnt_p = _k5_pool(zdrep, disrep[:, 0], batch_p, src, dst, zrow)
    return _k6(bins_p, cnt_p, b2r, Wl, blr)


# speedup vs baseline: 1.2898x; 1.2898x over previous
import functools

import jax
import jax.numpy as jnp
from jax import lax
from jax.experimental import pallas as pl
from jax.experimental.pallas import tpu as pltpu
from jax.experimental.pallas import tpu_sc as plsc

NN = 10000
EE = 320000
DD = 128
HH = 32
GG = 64
NP = 10240
CH = 128
SB = 4
EP = 327680
ER = EP // CH
TT = 10
NWORK = 32
PERT = NP // 16

_mesh = plsc.VectorSubcoreMesh(core_axis_name="c", subcore_axis_name="s")


def _wid(c, s):
    return s * 2 + c


@functools.partial(
    pl.kernel,
    out_type=jax.ShapeDtypeStruct((2, NP), jnp.float32),
    mesh=_mesh,
    scratch_types=[
        pltpu.VMEM((2 * SB, CH), jnp.int32),
        pltpu.VMEM((CH,), jnp.float32),
        pltpu.VMEM((PERT,), jnp.float32),
        pltpu.SemaphoreType.DMA,
        pltpu.SemaphoreType.DMA,
        pltpu.VMEM_SHARED((NP,), jnp.float32),
    ],
)
def _k1_deg(dst2_hbm, z640_hbm, deg_out, idx_d, ones_v, zb, sem_i, sem_w, deg_s):
    c = lax.axis_index("c")
    s = lax.axis_index("s")
    wid = _wid(c, s)
    for i in range(CH // 16):
        ones_v[pl.ds(16 * i, 16)] = jnp.full((16,), 1.0, jnp.float32)
    pltpu.sync_copy(z640_hbm, zb)
    pltpu.sync_copy(zb, deg_s.at[pl.ds(s * PERT, PERT)])
    plsc.subcore_barrier()

    def body(t, carry):
        rowA = (wid + NWORK * (2 * t)) * SB
        rowB = (wid + NWORK * (2 * t + 1)) * SB
        d1 = pltpu.async_copy(dst2_hbm.at[pl.ds(rowA, SB)],
                              idx_d.at[pl.ds(0, SB)], sem_i)
        d2 = pltpu.async_copy(dst2_hbm.at[pl.ds(rowB, SB)],
                              idx_d.at[pl.ds(SB, SB)], sem_i)
        d1.wait()
        d2.wait()
        ws = [pltpu.async_copy(ones_v, deg_s.at[idx_d.at[q]], sem_w, add=True)
              for q in range(2 * SB)]
        for w in ws:
            w.wait()
        return carry

    lax.fori_loop(0, TT, body, 0)
    plsc.subcore_barrier()
    pltpu.sync_copy(deg_s.at[pl.ds(s * PERT, PERT)],
                    deg_out.at[c].at[pl.ds(s * PERT, PERT)])


def _k2a_body(d0_ref, d1_ref, disrepT_ref):
    deg = d0_ref[0] + d1_ref[0] + 1.0
    dis = 1.0 / jnp.sqrt(deg)
    disrepT_ref[...] = jnp.broadcast_to(dis, (HH, dis.shape[1]))


def _k2a(deg_p):
    rb = 1024
    grid = NP // rb
    deg3 = deg_p.reshape(2 * NP // rb, 1, rb)
    return pl.pallas_call(
        _k2a_body,
        grid=(grid,),
        in_specs=[
            pl.BlockSpec((1, 1, rb), lambda i: (i, 0, 0)),
            pl.BlockSpec((1, 1, rb), lambda i: (i + NP // rb, 0, 0)),
        ],
        out_specs=pl.BlockSpec((HH, rb), lambda i: (0, i)),
        out_shape=jax.ShapeDtypeStruct((HH, NP), jnp.float32),
    )(deg3, deg3)


def _k2b_body(x_ref, w1_ref, disrep_ref, hd_ref):
    h = jnp.dot(x_ref[...], w1_ref[...], preferred_element_type=jnp.float32)
    hd_ref[...] = h * disrep_ref[...]


def _k2b(x_p, w1, disrep):
    rb = 1024
    grid = NP // rb
    return pl.pallas_call(
        _k2b_body,
        grid=(grid,),
        in_specs=[
            pl.BlockSpec((rb, DD), lambda i: (i, 0)),
            pl.BlockSpec((DD, HH), lambda i: (0, 0)),
            pl.BlockSpec((rb, HH), lambda i: (i, 0)),
        ],
        out_specs=pl.BlockSpec((rb, HH), lambda i: (i, 0)),
        out_shape=jax.ShapeDtypeStruct((NP, HH), jnp.float32),
    )(x_p, w1, disrep)


@functools.partial(
    pl.kernel,
    out_type=jax.ShapeDtypeStruct((2, NP, HH), jnp.float32),
    mesh=_mesh,
    scratch_types=[
        pltpu.VMEM((2 * SB, CH), jnp.int32),
        pltpu.VMEM((2 * SB, CH), jnp.int32),
        pltpu.VMEM((2 * SB * CH, HH), jnp.float32),
        pltpu.VMEM((CH, HH), jnp.float32),
        pltpu.SemaphoreType.DMA,
        pltpu.SemaphoreType.DMA,
        pltpu.SemaphoreType.DMA,
        pltpu.VMEM_SHARED((NP, HH), jnp.float32),
    ],
    compiler_params=pltpu.CompilerParams(use_tc_tiling_on_sc=False),
)
def _k3_msg(hd_hbm, src2_hbm, dst2_hbm, zrow_hbm, acc_out,
            idx_s, idx_d, rows, zb, sem_i, sem_g, sem_w, acc_s):
    c = lax.axis_index("c")
    s = lax.axis_index("s")
    wid = _wid(c, s)
    pltpu.sync_copy(zrow_hbm, zb)
    for j in range(PERT // CH):
        pltpu.sync_copy(zb, acc_s.at[pl.ds(s * PERT + j * CH, CH)])
    plsc.subcore_barrier()

    def body(t, carry):
        rowA = (wid + NWORK * (2 * t)) * SB
        rowB = (wid + NWORK * (2 * t + 1)) * SB
        ds_ = [
            pltpu.async_copy(src2_hbm.at[pl.ds(rowA, SB)],
                             idx_s.at[pl.ds(0, SB)], sem_i),
            pltpu.async_copy(dst2_hbm.at[pl.ds(rowA, SB)],
                             idx_d.at[pl.ds(0, SB)], sem_i),
            pltpu.async_copy(src2_hbm.at[pl.ds(rowB, SB)],
                             idx_s.at[pl.ds(SB, SB)], sem_i),
            pltpu.async_copy(dst2_hbm.at[pl.ds(rowB, SB)],
                             idx_d.at[pl.ds(SB, SB)], sem_i),
        ]
        for d in ds_:
            d.wait()
        gs = [pltpu.async_copy(hd_hbm.at[idx_s.at[q]],
                               rows.at[pl.ds(CH * q, CH)], sem_g)
              for q in range(2 * SB)]
        for g in gs:
            g.wait()
        ws = [pltpu.async_copy(rows.at[pl.ds(CH * q, CH)],
                               acc_s.at[idx_d.at[q]], sem_w, add=True)
              for q in range(2 * SB)]
        for w in ws:
            w.wait()
        return carry

    lax.fori_loop(0, TT, body, 0)
    plsc.subcore_barrier()
    pltpu.sync_copy(acc_s.at[pl.ds(s * PERT, PERT)],
                    acc_out.at[c].at[pl.ds(s * PERT, PERT)])


def _k4_body(acc_ref, hd_ref, disrep_ref, b1_ref, w2_ref, zd_ref):
    disrep = disrep_ref[...]
    pre = (acc_ref[0] + acc_ref[1] + hd_ref[...]) * disrep + b1_ref[...]
    a1 = jnp.maximum(pre, 0.0)
    z = jnp.dot(a1, w2_ref[...], preferred_element_type=jnp.float32)
    zd_ref[...] = z * disrep


def _k4(acc_p, hd, disrep, b1r, w2):
    rb = 1024
    grid = NP // rb
    return pl.pallas_call(
        _k4_body,
        grid=(grid,),
        in_specs=[
            pl.BlockSpec((2, rb, HH), lambda i: (0, i, 0)),
            pl.BlockSpec((rb, HH), lambda i: (i, 0)),
            pl.BlockSpec((rb, HH), lambda i: (i, 0)),
            pl.BlockSpec((1, HH), lambda i: (0, 0)),
            pl.BlockSpec((HH, HH), lambda i: (0, 0)),
        ],
        out_specs=pl.BlockSpec((rb, HH), lambda i: (i, 0)),
        out_shape=jax.ShapeDtypeStruct((NP, HH), jnp.float32),
    )(acc_p, hd, disrep, b1r, w2)


@functools.partial(
    pl.kernel,
    out_type=(jax.ShapeDtypeStruct((2, CH, HH), jnp.float32),
              jax.ShapeDtypeStruct((2, CH, HH), jnp.float32)),
    mesh=_mesh,
    scratch_types=[
        pltpu.VMEM((2 * SB, CH), jnp.int32),
        pltpu.VMEM((2 * SB, CH), jnp.int32),
        pltpu.VMEM((2 * SB * CH, HH), jnp.float32),
        pltpu.VMEM((CH, HH), jnp.float32),
        pltpu.VMEM((CH, HH), jnp.float32),
        pltpu.VMEM((CH, HH), jnp.float32),
        pltpu.VMEM((CH, HH), jnp.float32),
        pltpu.VMEM((CH,), jnp.float32),
        pltpu.VMEM((CH,), jnp.int32),
        pltpu.VMEM((CH, HH), jnp.float32),
        pltpu.SemaphoreType.DMA,
        pltpu.SemaphoreType.DMA,
        pltpu.SemaphoreType.DMA,
        pltpu.VMEM_SHARED((NP, HH), jnp.float32),
        pltpu.VMEM_SHARED((CH, HH), jnp.float32),
        pltpu.VMEM_SHARED((CH, HH), jnp.float32),
    ],
    compiler_params=pltpu.CompilerParams(
        use_tc_tiling_on_sc=False, needs_layout_passes=False),
)
def _k5_pool(zd_hbm, dis_hbm, batch_hbm, src2_hbm, dst2_hbm, zrow_hbm,
             bins_out, cnt_out,
             idx_s, idx_d, rows, rbuf2, zbuf2, qbuf2, cbuf2, dbuf, bbuf, zb,
             sem_i, sem_g, sem_w, racc_s, bins_s, cbin_s):
    c = lax.axis_index("c")
    s = lax.axis_index("s")
    wid = _wid(c, s)
    pltpu.sync_copy(zrow_hbm, zb)
    for j in range(PERT // CH):
        pltpu.sync_copy(zb, racc_s.at[pl.ds(s * PERT + j * CH, CH)])

    @pl.when(s == 0)
    def _():
        pltpu.sync_copy(zb, bins_s)

    @pl.when(s == 1)
    def _():
        pltpu.sync_copy(zb, cbin_s)

    plsc.subcore_barrier()

    def body(t, carry):
        rowA = (wid + NWORK * (2 * t)) * SB
        rowB = (wid + NWORK * (2 * t + 1)) * SB
        ds_ = [
            pltpu.async_copy(src2_hbm.at[pl.ds(rowA, SB)],
                             idx_s.at[pl.ds(0, SB)], sem_i),
            pltpu.async_copy(dst2_hbm.at[pl.ds(rowA, SB)],
                             idx_d.at[pl.ds(0, SB)], sem_i),
            pltpu.async_copy(src2_hbm.at[pl.ds(rowB, SB)],
                             idx_s.at[pl.ds(SB, SB)], sem_i),
            pltpu.async_copy(dst2_hbm.at[pl.ds(rowB, SB)],
                             idx_d.at[pl.ds(SB, SB)], sem_i),
        ]
        for d in ds_:
            d.wait()
        gs = [pltpu.async_copy(zd_hbm.at[idx_s.at[q]],
                               rows.at[pl.ds(CH * q, CH)], sem_g)
              for q in range(2 * SB)]
        for g in gs:
            g.wait()
        ws = [pltpu.async_copy(rows.at[pl.ds(CH * q, CH)],
                               racc_s.at[idx_d.at[q]], sem_w, add=True)
              for q in range(2 * SB)]
        for w in ws:
            w.wait()
        return carry

    lax.fori_loop(0, TT, body, 0)
    plsc.subcore_barrier()

    coef = jnp.where(c == 0, 1.0, 0.0).astype(jnp.float32)

    def cfill(v, carry):
        for half in range(HH // 16):
            cbuf2[v, pl.ds(16 * half, 16)] = jnp.zeros((16,), jnp.float32) + coef
        return carry

    lax.fori_loop(0, CH, cfill, 0)

    for k in range(PERT // CH):
        off = s * PERT + k * CH
        pltpu.sync_copy(racc_s.at[pl.ds(off, CH)], rbuf2)
        pltpu.sync_copy(zd_hbm.at[pl.ds(off, CH)], zbuf2)
        pltpu.sync_copy(dis_hbm.at[pl.ds(off, CH)], dbuf)
        pltpu.sync_copy(batch_hbm.at[pl.ds(off, CH)], bbuf)

        def nbody(v, carry):
            dv = plsc.load_gather(dbuf, [jnp.zeros((16,), jnp.int32) + v])
            for half in range(HH // 16):
                sl = pl.ds(16 * half, 16)
                qbuf2[v, sl] = dv * (rbuf2[v, sl] + coef * zbuf2[v, sl])
            return carry

        lax.fori_loop(0, CH, nbody, 0)
        pltpu.sync_copy(qbuf2, bins_s.at[bbuf], add=True)
        pltpu.sync_copy(cbuf2, cbin_s.at[bbuf], add=True)
    plsc.subcore_barrier()

    @pl.when(s == 0)
    def _():
        pltpu.sync_copy(bins_s, bins_out.at[c])

    @pl.when(s == 1)
    def _():
        pltpu.sync_copy(cbin_s, cnt_out.at[c])


def _k6_body(bins_ref, cnt_ref, b2_ref, wl_ref, bl_ref, out_ref):
    b = bins_ref[0, :GG] + bins_ref[1, :GG]
    cn = cnt_ref[0, :GG] + cnt_ref[1, :GG]
    pooled = b / jnp.maximum(cn, 1.0) + jnp.where(cn > 0.0, b2_ref[...], 0.0)
    out = jnp.dot(pooled, wl_ref[...], preferred_element_type=jnp.float32)
    out_ref[...] = out + bl_ref[...]


def _k6(bins_p, cnt_p, b2r, wl, blr):
    return pl.pallas_call(
        _k6_body,
        in_specs=[
            pl.BlockSpec((2, CH, HH), lambda: (0, 0, 0)),
            pl.BlockSpec((2, CH, HH), lambda: (0, 0, 0)),
            pl.BlockSpec((1, HH), lambda: (0, 0)),
            pl.BlockSpec((HH, 1), lambda: (0, 0)),
            pl.BlockSpec((1, 1), lambda: (0, 0)),
        ],
        out_specs=pl.BlockSpec((GG, 1), lambda: (0, 0)),
        out_shape=jax.ShapeDtypeStruct((GG, 1), jnp.float32),
    )(bins_p, cnt_p, b2r, wl, blr)


def kernel(x, edge_index, batch, W1, b1, W2, b2, Wl, bl):
    src2 = jnp.pad(edge_index[0], (0, EP - EE), constant_values=NN).reshape(ER, CH)
    dst2 = jnp.pad(edge_index[1], (0, EP - EE), constant_values=NN).reshape(ER, CH)
    x_p = jnp.pad(x, ((0, NP - NN), (0, 0)))
    batch_p = jnp.pad(batch, (0, NP - NN), constant_values=GG)
    z640 = jnp.zeros((PERT,), jnp.float32)
    zrow = jnp.zeros((CH, HH), jnp.float32)
    b1r = b1.reshape(1, HH)
    b2r = b2.reshape(1, HH)
    blr = bl.reshape(1, 1)

    deg_p = _k1_deg(dst2, z640)
    disrep = _k2a(deg_p).T
    hd = _k2b(x_p, W1, disrep)
    acc_p = _k3_msg(hd, src2, dst2, zrow)
    zdrep = _k4(acc_p, hd, disrep, b1r, W2)
    bins_p, cnt_p = _k5_pool(zdrep, disrep[:, 0], batch_p, src2, dst2, zrow)
    return _k6(bins_p, cnt_p, b2r, Wl, blr)
